# baseline (speedup 1.0000x reference)
import jax
import jax.numpy as jnp
from jax import lax
from jax.experimental import pallas as pl
from jax.experimental.pallas import tpu as pltpu
from jax.experimental.pallas import tpu_sc as plsc

SEQ = 2048
N_HEAD = 32
BR = 1024
DEMO_PAD = 64
LANES = 16


def _sc_gather_scale_body(attn_flat, fin_hbm, cls_hbm, w_hbm, vals_hbm,
                          fin_v, cls_v, w_v, idx_v, gath_v, val_v, sem):
    c = lax.axis_index("c")
    s = lax.axis_index("s")
    h = s * 2 + c
    pltpu.sync_copy(fin_hbm, fin_v)
    pltpu.sync_copy(cls_hbm, cls_v)
    pltpu.sync_copy(w_hbm.at[h], w_v)
    for j in range(DEMO_PAD // LANES):
        sl = pl.ds(j * LANES, LANES)
        idx_v[sl] = h * (SEQ * SEQ) + fin_v[sl] * SEQ + cls_v[sl]
    pltpu.async_copy(attn_flat.at[idx_v], gath_v, sem).wait()
    for j in range(DEMO_PAD // LANES):
        sl = pl.ds(j * LANES, LANES)
        val_v[sl] = gath_v[sl] * jnp.exp(w_v[sl])
    pltpu.sync_copy(val_v, vals_hbm.at[h])


def _sc_gather_scale(attn_flat, fin_pad, cls_pad, wpad):
    mesh = plsc.VectorSubcoreMesh(core_axis_name="c", subcore_axis_name="s")
    return pl.kernel(
        _sc_gather_scale_body,
        out_type=jax.ShapeDtypeStruct((N_HEAD, DEMO_PAD), jnp.float32),
        mesh=mesh,
        scratch_types=[
            pltpu.VMEM((DEMO_PAD,), jnp.int32),
            pltpu.VMEM((DEMO_PAD,), jnp.int32),
            pltpu.VMEM((DEMO_PAD,), jnp.float32),
            pltpu.VMEM((DEMO_PAD,), jnp.int32),
            pltpu.VMEM((DEMO_PAD,), jnp.float32),
            pltpu.VMEM((DEMO_PAD,), jnp.float32),
            pltpu.SemaphoreType.DMA,
        ],
    )(attn_flat, fin_pad, cls_pad, wpad)


def _copy_fixup_body(final_ref, class_ref, attn_ref, vals_ref, out_ref):
    rb = pl.program_id(1)
    out_ref[...] = attn_ref[...]
    base = rb * BR
    vrow = vals_ref[0, :, :]
    col = lax.broadcasted_iota(jnp.int32, (1, SEQ), 1)
    n_demo = final_ref.shape[0]
    for d in range(n_demo):
        f = final_ref[d]
        c = class_ref[d]

        @pl.when((f >= base) & (f < base + BR))
        def _(d=d, f=f, c=c):
            rl = f - base
            cur = out_ref[0, 0, pl.ds(rl, 1), :]
            vd = vrow[:, d:d + 1]
            out_ref[0, 0, pl.ds(rl, 1), :] = jnp.where(col == c, vd, cur)


def kernel(attn_weights, class_poss, final_poss, weight):
    n_head = attn_weights.shape[1]
    seq = attn_weights.shape[2]
    n_demo = class_poss.shape[0]
    pad = DEMO_PAD - n_demo
    fin_pad = jnp.concatenate(
        [final_poss, jnp.broadcast_to(final_poss[-1:], (pad,))])
    cls_pad = jnp.concatenate(
        [class_poss, jnp.broadcast_to(class_poss[-1:], (pad,))])
    wpad = jnp.zeros((n_head, DEMO_PAD), jnp.float32).at[:, :n_demo].set(weight)

    vals = _sc_gather_scale(attn_weights.reshape(-1), fin_pad, cls_pad, wpad)

    grid_spec = pltpu.PrefetchScalarGridSpec(
        num_scalar_prefetch=2,
        grid=(n_head, seq // BR),
        in_specs=[
            pl.BlockSpec((1, 1, BR, seq), lambda h, rb, *_: (0, h, rb, 0)),
            pl.BlockSpec((1, 1, DEMO_PAD), lambda h, rb, *_: (h, 0, 0)),
        ],
        out_specs=pl.BlockSpec((1, 1, BR, seq), lambda h, rb, *_: (0, h, rb, 0)),
    )
    return pl.pallas_call(
        _copy_fixup_body,
        grid_spec=grid_spec,
        out_shape=jax.ShapeDtypeStruct(attn_weights.shape, attn_weights.dtype),
    )(final_poss, class_poss, attn_weights, vals.reshape(n_head, 1, DEMO_PAD))


# speedup vs baseline: 1.9431x; 1.9431x over previous
import jax
import jax.numpy as jnp
from jax import lax
from jax.experimental import pallas as pl
from jax.experimental.pallas import tpu as pltpu
from jax.experimental.pallas import tpu_sc as plsc

SEQ = 2048
N_HEAD = 32
BR = 1024
DEMO_PAD = 64
LANES = 16
ROWS_PER_BATCH = 16


def _sc_row_gather_body(attn2d, fin_hbm, rows_hbm, fin_v, idx_v, rows_v, sem):
    c = lax.axis_index("c")
    s = lax.axis_index("s")
    h = s * 2 + c
    pltpu.sync_copy(fin_hbm, fin_v)
    for j in range(DEMO_PAD // LANES):
        sl = pl.ds(j * LANES, LANES)
        idx_v[sl] = h * SEQ + fin_v[sl]
    for b in range(DEMO_PAD // ROWS_PER_BATCH):
        pltpu.async_copy(
            attn2d.at[idx_v[pl.ds(b * ROWS_PER_BATCH, ROWS_PER_BATCH)]],
            rows_v, sem).wait()
        pltpu.sync_copy(
            rows_v,
            rows_hbm.at[pl.ds(h * DEMO_PAD + b * ROWS_PER_BATCH,
                              ROWS_PER_BATCH)])


def _sc_row_gather(attn2d, fin_pad):
    mesh = plsc.VectorSubcoreMesh(core_axis_name="c", subcore_axis_name="s")
    return pl.kernel(
        _sc_row_gather_body,
        out_type=jax.ShapeDtypeStruct((N_HEAD * DEMO_PAD, SEQ), jnp.float32),
        mesh=mesh,
        compiler_params=pltpu.CompilerParams(use_tc_tiling_on_sc=True),
        scratch_types=[
            pltpu.VMEM((DEMO_PAD,), jnp.int32),
            pltpu.VMEM((DEMO_PAD,), jnp.int32),
            pltpu.VMEM((ROWS_PER_BATCH, SEQ), jnp.float32),
            pltpu.SemaphoreType.DMA,
        ],
    )(attn2d, fin_pad)


def _copy_fixup_body(final_ref, class_ref, attn_ref, rows_ref, w_ref, out_ref):
    rb = pl.program_id(1)
    out_ref[...] = attn_ref[...]
    base = rb * BR
    wrow = jnp.exp(w_ref[0, :, :])
    col = lax.broadcasted_iota(jnp.int32, (1, SEQ), 1)
    n_demo = final_ref.shape[0]
    for d in range(n_demo):
        f = final_ref[d]
        c = class_ref[d]

        @pl.when((f >= base) & (f < base + BR))
        def _(d=d, f=f, c=c):
            rl = f - base
            cur = out_ref[0, 0, pl.ds(rl, 1), :]
            vd = rows_ref[d:d + 1, :] * wrow[:, d:d + 1]
            out_ref[0, 0, pl.ds(rl, 1), :] = jnp.where(col == c, vd, cur)


def kernel(attn_weights, class_poss, final_poss, weight):
    n_head = attn_weights.shape[1]
    seq = attn_weights.shape[2]
    n_demo = class_poss.shape[0]
    pad = DEMO_PAD - n_demo
    fin_pad = jnp.concatenate(
        [final_poss, jnp.broadcast_to(final_poss[-1:], (pad,))])
    wpad = jnp.zeros((n_head, 1, DEMO_PAD), jnp.float32)
    wpad = wpad.at[:, 0, :n_demo].set(weight)

    rows = _sc_row_gather(attn_weights.reshape(n_head * seq, seq), fin_pad)

    grid_spec = pltpu.PrefetchScalarGridSpec(
        num_scalar_prefetch=2,
        grid=(n_head, seq // BR),
        in_specs=[
            pl.BlockSpec((1, 1, BR, seq), lambda h, rb, *_: (0, h, rb, 0)),
            pl.BlockSpec((DEMO_PAD, seq), lambda h, rb, *_: (h, 0)),
            pl.BlockSpec((1, 1, DEMO_PAD), lambda h, rb, *_: (h, 0, 0)),
        ],
        out_specs=pl.BlockSpec((1, 1, BR, seq), lambda h, rb, *_: (0, h, rb, 0)),
    )
    return pl.pallas_call(
        _copy_fixup_body,
        grid_spec=grid_spec,
        out_shape=jax.ShapeDtypeStruct(attn_weights.shape, attn_weights.dtype),
    )(final_poss, class_poss, attn_weights, rows, wpad)


# speedup vs baseline: 1.9539x; 1.0055x over previous
import jax
import jax.numpy as jnp
from jax import lax
from jax.experimental import pallas as pl
from jax.experimental.pallas import tpu as pltpu
from jax.experimental.pallas import tpu_sc as plsc

SEQ = 2048
N_HEAD = 32
BR = 1024
DEMO_PAD = 64
LANES = 16
ROWS_PER_BATCH = 16


def _sc_row_gather_body(attn2d, fin_hbm, rows_hbm, fin_v, idx_v,
                        buf0, buf1, gs0, gs1, ws0, ws1):
    c = lax.axis_index("c")
    s = lax.axis_index("s")
    h = s * 2 + c
    pltpu.sync_copy(fin_hbm, fin_v)
    for j in range(DEMO_PAD // LANES):
        sl = pl.ds(j * LANES, LANES)
        idx_v[sl] = h * SEQ + fin_v[sl]
    bufs = (buf0, buf1)
    gsems = (gs0, gs1)
    wsems = (ws0, ws1)
    n_batch = DEMO_PAD // ROWS_PER_BATCH

    def gather(b):
        return pltpu.async_copy(
            attn2d.at[idx_v[pl.ds(b * ROWS_PER_BATCH, ROWS_PER_BATCH)]],
            bufs[b % 2], gsems[b % 2])

    def write(b):
        return pltpu.async_copy(
            bufs[b % 2],
            rows_hbm.at[pl.ds(h * DEMO_PAD + b * ROWS_PER_BATCH,
                              ROWS_PER_BATCH)],
            wsems[b % 2])

    gh = {0: gather(0)}
    wh = {}
    for b in range(n_batch):
        gh[b].wait()
        wh[b] = write(b)
        nb = b + 1
        if nb < n_batch:
            if nb >= 2:
                wh[nb - 2].wait()
            gh[nb] = gather(nb)
    wh[n_batch - 2].wait()
    wh[n_batch - 1].wait()


def _sc_row_gather(attn2d, fin_pad):
    mesh = plsc.VectorSubcoreMesh(core_axis_name="c", subcore_axis_name="s")
    return pl.kernel(
        _sc_row_gather_body,
        out_type=jax.ShapeDtypeStruct((N_HEAD * DEMO_PAD, SEQ), jnp.float32),
        mesh=mesh,
        compiler_params=pltpu.CompilerParams(use_tc_tiling_on_sc=True),
        scratch_types=[
            pltpu.VMEM((DEMO_PAD,), jnp.int32),
            pltpu.VMEM((DEMO_PAD,), jnp.int32),
            pltpu.VMEM((ROWS_PER_BATCH, SEQ), jnp.float32),
            pltpu.VMEM((ROWS_PER_BATCH, SEQ), jnp.float32),
            pltpu.SemaphoreType.DMA,
            pltpu.SemaphoreType.DMA,
            pltpu.SemaphoreType.DMA,
            pltpu.SemaphoreType.DMA,
        ],
    )(attn2d, fin_pad)


def _copy_fixup_body(final_ref, class_ref, attn_ref, rows_ref, w_ref, out_ref):
    rb = pl.program_id(1)
    out_ref[...] = attn_ref[...]
    base = rb * BR
    wrow = jnp.exp(w_ref[0, :, :])
    col = lax.broadcasted_iota(jnp.int32, (1, SEQ), 1)
    n_demo = final_ref.shape[0]
    for d in range(n_demo):
        f = final_ref[d]
        c = class_ref[d]

        @pl.when((f >= base) & (f < base + BR))
        def _(d=d, f=f, c=c):
            rl = f - base
            cur = out_ref[0, 0, pl.ds(rl, 1), :]
            vd = rows_ref[d:d + 1, :] * wrow[:, d:d + 1]
            out_ref[0, 0, pl.ds(rl, 1), :] = jnp.where(col == c, vd, cur)


def kernel(attn_weights, class_poss, final_poss, weight):
    n_head = attn_weights.shape[1]
    seq = attn_weights.shape[2]
    n_demo = class_poss.shape[0]
    pad = DEMO_PAD - n_demo
    fin_pad = jnp.concatenate(
        [final_poss, jnp.broadcast_to(final_poss[-1:], (pad,))])
    wpad = jnp.zeros((n_head, 1, DEMO_PAD), jnp.float32)
    wpad = wpad.at[:, 0, :n_demo].set(weight)

    rows = _sc_row_gather(attn_weights.reshape(n_head * seq, seq), fin_pad)

    grid_spec = pltpu.PrefetchScalarGridSpec(
        num_scalar_prefetch=2,
        grid=(n_head, seq // BR),
        in_specs=[
            pl.BlockSpec((1, 1, BR, seq), lambda h, rb, *_: (0, h, rb, 0)),
            pl.BlockSpec((DEMO_PAD, seq), lambda h, rb, *_: (h, 0)),
            pl.BlockSpec((1, 1, DEMO_PAD), lambda h, rb, *_: (h, 0, 0)),
        ],
        out_specs=pl.BlockSpec((1, 1, BR, seq), lambda h, rb, *_: (0, h, rb, 0)),
    )
    return pl.pallas_call(
        _copy_fixup_body,
        grid_spec=grid_spec,
        out_shape=jax.ShapeDtypeStruct(attn_weights.shape, attn_weights.dtype),
    )(final_poss, class_poss, attn_weights, rows, wpad)
